# baseline (speedup 1.0000x reference)
import functools

import jax
import jax.numpy as jnp
from jax import lax
from jax.experimental import pallas as pl
from jax.experimental.pallas import tpu as pltpu
from jax.experimental.pallas import tpu_sc as plsc

_N = 10000
_NP = 10240
_E = 320000
_D = 128
_D_OUT = 64
_B = 4096

_INFO = plsc.get_sparse_core_info()
_NC, _NS = _INFO.num_cores, _INFO.num_subcores
_NW = _NC * _NS
_MESH = plsc.VectorSubcoreMesh(core_axis_name="c", subcore_axis_name="s")


_EPT = _NP
_K = 64
_CH = _EPT // _K
_HD = _D // 2


def _make_deg():
    k = _K
    rpt = _CH
    wpt = _NP // _NS

    @functools.partial(
        pl.kernel,
        out_type=(jax.ShapeDtypeStruct((_NP,), jnp.float32),
                  jax.ShapeDtypeStruct((_NP,), jnp.float32)),
        mesh=_MESH,
        compiler_params=pltpu.CompilerParams(use_tc_tiling_on_sc=True),
        scratch_types=[
            pltpu.VMEM((rpt, k), jnp.int32),
            pltpu.VMEM((k,), jnp.float32),
            pltpu.VMEM((wpt,), jnp.float32),
            pltpu.VMEM_SHARED((_NP,), jnp.float32),
            pltpu.SemaphoreType.DMA,
        ],
    )
    def deg_kernel(dst_hbm, out0_hbm, out1_hbm, idx_v, ones_v, zero_v, acc,
                   sem):
        c = lax.axis_index("c")
        s = lax.axis_index("s")
        wid = c * _NS + s
        pltpu.sync_copy(dst_hbm.at[wid], idx_v)

        def fill16(j, carry):
            ones_v[pl.ds(j * 16, 16)] = jnp.ones((16,), jnp.float32)
            return carry

        lax.fori_loop(0, k // 16, fill16, 0)

        def zfill(j, carry):
            zero_v[pl.ds(j * 16, 16)] = jnp.zeros((16,), jnp.float32)
            return carry

        lax.fori_loop(0, wpt // 16, zfill, 0)
        pltpu.sync_copy(zero_v, acc.at[pl.ds(s * wpt, wpt)])
        plsc.subcore_barrier()

        def body(i, carry):
            pltpu.async_copy(ones_v, acc.at[idx_v.at[i]], sem, add=True)
            return carry

        lax.fori_loop(0, rpt, body, 0)

        def drain(i, carry):
            pltpu.make_async_copy(ones_v, acc.at[idx_v.at[0]], sem).wait()
            return carry

        lax.fori_loop(0, rpt, drain, 0)
        plsc.subcore_barrier()

        @pl.when(c == 0)
        def _():
            pltpu.sync_copy(acc.at[pl.ds(s * wpt, wpt)],
                            out0_hbm.at[pl.ds(s * wpt, wpt)])

        @pl.when(c == 1)
        def _():
            pltpu.sync_copy(acc.at[pl.ds(s * wpt, wpt)],
                            out1_hbm.at[pl.ds(s * wpt, wpt)])

    return deg_kernel


_deg = _make_deg()


def _scale_body(deg0_ref, deg1_ref, x_ref, w_ref, out_ref):
    deg = deg0_ref[:, 0] + deg1_ref[:, 0] + 1.0
    dinv = lax.rsqrt(jnp.maximum(deg, 1e-12))
    h = jnp.dot(x_ref[...], w_ref[...], preferred_element_type=jnp.float32)
    row = (lax.broadcasted_iota(jnp.int32, (_BLK, 1), 0)
           + pl.program_id(0) * _BLK)
    out_ref[...] = jnp.where(row < _N, h * dinv[:, None], 0.0)


_BLK = 1024


def _scale(deg0, deg1, x, w):
    grid = _NP // _BLK
    return pl.pallas_call(
        _scale_body,
        out_shape=jax.ShapeDtypeStruct((_NP, _D), jnp.float32),
        grid=(grid,),
        in_specs=[
            pl.BlockSpec((_BLK, 1), lambda i: (i, 0)),
            pl.BlockSpec((_BLK, 1), lambda i: (i, 0)),
            pl.BlockSpec((_BLK, _D), lambda i: (i, 0)),
            pl.BlockSpec((_D, _D), lambda i: (0, 0)),
        ],
        out_specs=pl.BlockSpec((_BLK, _D), lambda i: (i, 0)),
    )(deg0, deg1, x, w)


def _make_edge_agg():
    k = _K
    ch = _CH
    nbuf = 5
    gen = ch // nbuf
    iw = 16
    spt = _NP // _NS

    @functools.partial(
        pl.kernel,
        out_type=jax.ShapeDtypeStruct((_NC, _NP, _D), jnp.float32),
        mesh=_MESH,
        compiler_params=pltpu.CompilerParams(use_tc_tiling_on_sc=True),
        scratch_types=[
            pltpu.VMEM((2, iw, k), jnp.int32),
            pltpu.VMEM((iw, k), jnp.int32),
            [pltpu.VMEM((k, _D), jnp.float32)] * nbuf,
            pltpu.VMEM_SHARED((_NP, _D), jnp.float32),
            [pltpu.SemaphoreType.DMA] * nbuf,
            [pltpu.SemaphoreType.DMA] * nbuf,
        ],
    )
    def edge_agg(scaled_hbm, src_hbm, dst_hbm, out_hbm, src_v, dst_v, rows_v,
                 acc, gsems, ssems):
        c = lax.axis_index("c")
        s = lax.axis_index("s")
        wid = c * _NS + s
        pltpu.sync_copy(src_hbm.at[wid, pl.ds(0, iw)], src_v.at[0])
        pltpu.sync_copy(dst_hbm.at[wid, pl.ds(0, iw)], dst_v)
        pltpu.sync_copy(scaled_hbm.at[pl.ds(s * spt, spt)],
                        acc.at[pl.ds(s * spt, spt)])
        for b in range(nbuf - 1):
            pltpu.async_copy(scaled_hbm.at[src_v.at[0, b]], rows_v[b],
                             gsems[b])
        plsc.subcore_barrier()

        def gwait(b):
            pltpu.make_async_copy(scaled_hbm.at[src_v.at[0, 0]],
                                  rows_v[b], gsems[b]).wait()

        def swait(b):
            pltpu.make_async_copy(rows_v[b], acc.at[dst_v.at[0]],
                                  ssems[b]).wait()

        def body(g, carry):
            for b in range(nbuf):
                i = g * nbuf + b
                pb = (b + nbuf - 1) % nbuf
                gwait(b)

                @pl.when(i > 0)
                def _():
                    swait(pb)

                @pl.when((i % iw == 0) & (i > 0))
                def _():
                    off = pl.multiple_of(i, iw)
                    pltpu.sync_copy(dst_hbm.at[wid, pl.ds(off, iw)], dst_v)

                pltpu.async_copy(rows_v[b], acc.at[dst_v.at[i % iw]],
                                 ssems[b], add=True)
                j = i + nbuf - 1

                @pl.when(j < ch)
                def _():
                    w = (j // iw) % 2

                    @pl.when(j % iw == 0)
                    def _():
                        off = pl.multiple_of(j, iw)
                        pltpu.sync_copy(src_hbm.at[wid, pl.ds(off, iw)],
                                        src_v.at[w])

                    pltpu.async_copy(scaled_hbm.at[src_v.at[w, j % iw]],
                                     rows_v[pb], gsems[pb])

            return carry

        lax.fori_loop(0, gen, body, 0)
        swait((ch - 1) % nbuf)
        plsc.subcore_barrier()
        pltpu.sync_copy(acc.at[pl.ds(s * spt, spt)],
                        out_hbm.at[c, pl.ds(s * spt, spt)])

    return edge_agg


_edge_agg = _make_edge_agg()


def _combine_body(p_ref, scaled_ref, deg0_ref, deg1_ref, b_ref, out_ref):
    deg = deg0_ref[:, 0] + deg1_ref[:, 0] + 1.0
    dinv = lax.rsqrt(jnp.maximum(deg, 1e-12))
    msum = p_ref[0] + p_ref[1] - scaled_ref[...]
    out_ref[...] = jnp.maximum(dinv[:, None] * msum + b_ref[...], 0.0)


def _combine(partials, scaled_p, deg0, deg1, b_gcn2):
    grid = _NP // _BLK
    return pl.pallas_call(
        _combine_body,
        out_shape=jax.ShapeDtypeStruct((_NP, _D), jnp.float32),
        grid=(grid,),
        in_specs=[
            pl.BlockSpec((_NC, _BLK, _D), lambda i: (0, i, 0)),
            pl.BlockSpec((_BLK, _D), lambda i: (i, 0)),
            pl.BlockSpec((_BLK, 1), lambda i: (i, 0)),
            pl.BlockSpec((_BLK, 1), lambda i: (i, 0)),
            pl.BlockSpec((1, _D), lambda i: (0, 0)),
        ],
        out_specs=pl.BlockSpec((_BLK, _D), lambda i: (i, 0)),
    )(partials, scaled_p, deg0, deg1, b_gcn2)


def _make_gather():
    nb = 3 * _B
    k = 128
    rows_all = nb // k
    rpt = rows_all // _NW

    @functools.partial(
        pl.kernel,
        out_type=jax.ShapeDtypeStruct((rows_all, k, _D), jnp.float32),
        mesh=_MESH,
        compiler_params=pltpu.CompilerParams(use_tc_tiling_on_sc=True),
        scratch_types=[
            pltpu.VMEM((rpt, k), jnp.int32),
            pltpu.VMEM((rpt, k, _D), jnp.float32),
            pltpu.SemaphoreType.DMA,
        ],
    )
    def gather_kernel(gcn_hbm, idx_hbm, out_hbm, idx_v, rows_v, sem):
        c = lax.axis_index("c")
        s = lax.axis_index("s")
        wid = c * _NS + s
        pltpu.sync_copy(idx_hbm.at[wid], idx_v)
        for j in range(rpt):
            pltpu.async_copy(gcn_hbm.at[idx_v.at[j]], rows_v.at[j], sem)
        for j in range(rpt):
            pltpu.make_async_copy(gcn_hbm.at[idx_v.at[j]], rows_v.at[j],
                                  sem).wait()
        pltpu.sync_copy(rows_v, out_hbm.at[pl.ds(wid * rpt, rpt)])

    return gather_kernel


_gather = _make_gather()


def _mlp_loss_body(z_ref, w1_ref, b1_ref, w2_ref, b2_ref, out_ref):
    h1 = jnp.dot(z_ref[...], w1_ref[...], preferred_element_type=jnp.float32)
    h1 = jnp.maximum(h1 + b1_ref[...], 0.0)
    o = jnp.dot(h1, w2_ref[...], preferred_element_type=jnp.float32)
    o = o + b2_ref[...]
    a = o[0:_B]
    p = o[_B:2 * _B]
    n = o[2 * _B:3 * _B]
    eps = 1e-6
    d_ap = jnp.sqrt(jnp.sum((a - p + eps) ** 2, axis=1))
    d_an = jnp.sqrt(jnp.sum((a - n + eps) ** 2, axis=1))
    loss = jnp.mean(jnp.maximum(d_ap - d_an + 1.0, 0.0))
    out_ref[...] = loss.reshape(1, 1)


def _mlp_loss(z, w1, b1_2, w2, b2_2):
    return pl.pallas_call(
        _mlp_loss_body,
        out_shape=jax.ShapeDtypeStruct((1, 1), jnp.float32),
        in_specs=[
            pl.BlockSpec((3 * _B, _D), lambda: (0, 0)),
            pl.BlockSpec((_D, _D), lambda: (0, 0)),
            pl.BlockSpec((1, _D), lambda: (0, 0)),
            pl.BlockSpec((_D, _D_OUT), lambda: (0, 0)),
            pl.BlockSpec((1, _D_OUT), lambda: (0, 0)),
        ],
        out_specs=pl.BlockSpec((1, 1), lambda: (0, 0)),
    )(z, w1, b1_2, w2, b2_2)


def kernel(x, W_gcn, b_gcn, W1, b1, W2, b2, edge_index,
           anchor_idx, positive_idx, negative_idx):
    ept = _E // _NW
    srcm = edge_index[0].reshape(_NW, ept)
    dstm = edge_index[1].reshape(_NW, ept)
    pad = jnp.broadcast_to(
        jnp.arange(_N, _N + _EPT - ept, dtype=jnp.int32), (_NW, _EPT - ept))
    src4 = jnp.concatenate([srcm, pad], axis=1).reshape(_NW, _CH, _K)
    dst4 = jnp.concatenate([dstm, pad], axis=1).reshape(_NW, _CH, _K)
    deg0, deg1 = _deg(dst4)
    deg0 = deg0.reshape(_NP, 1)
    deg1 = deg1.reshape(_NP, 1)
    scaled_p = _scale(deg0, deg1, x, W_gcn)
    partials = _edge_agg(scaled_p, src4, dst4)
    gcn = _combine(partials, scaled_p, deg0, deg1, b_gcn.reshape(1, _D))
    idx3 = jnp.concatenate([anchor_idx, positive_idx, negative_idx])
    z = _gather(gcn, idx3.reshape(_NW, 3, 128)).reshape(3 * _B, _D)
    loss = _mlp_loss(z, W1, b1.reshape(1, _D), W2, b2.reshape(1, _D_OUT))
    return loss[0, 0]


# speedup vs baseline: 1.0187x; 1.0187x over previous
import functools

import jax
import jax.numpy as jnp
from jax import lax
from jax.experimental import pallas as pl
from jax.experimental.pallas import tpu as pltpu
from jax.experimental.pallas import tpu_sc as plsc

_N = 10000
_NP = 10240
_E = 320000
_D = 128
_D_OUT = 64
_B = 4096

_INFO = plsc.get_sparse_core_info()
_NC, _NS = _INFO.num_cores, _INFO.num_subcores
_NW = _NC * _NS
_MESH = plsc.VectorSubcoreMesh(core_axis_name="c", subcore_axis_name="s")


_EPT = _NP
_K = 64
_CH = _EPT // _K
_HD = _D // 2


def _make_deg():
    k = _K
    rpt = _CH
    wpt = _NP // _NS

    @functools.partial(
        pl.kernel,
        out_type=(jax.ShapeDtypeStruct((_NP,), jnp.float32),
                  jax.ShapeDtypeStruct((_NP,), jnp.float32)),
        mesh=_MESH,
        scratch_types=[
            pltpu.VMEM((rpt, k), jnp.int32),
            pltpu.VMEM((k,), jnp.float32),
            pltpu.VMEM((wpt,), jnp.float32),
            pltpu.VMEM_SHARED((_NP,), jnp.float32),
            pltpu.SemaphoreType.DMA,
        ],
    )
    def deg_kernel(dst_hbm, out0_hbm, out1_hbm, idx_v, ones_v, zero_v, acc,
                   sem):
        c = lax.axis_index("c")
        s = lax.axis_index("s")
        wid = c * _NS + s
        pltpu.sync_copy(dst_hbm.at[wid], idx_v)

        def fill16(j, carry):
            ones_v[pl.ds(j * 16, 16)] = jnp.ones((16,), jnp.float32)
            return carry

        lax.fori_loop(0, k // 16, fill16, 0)

        def zfill(j, carry):
            zero_v[pl.ds(j * 16, 16)] = jnp.zeros((16,), jnp.float32)
            return carry

        lax.fori_loop(0, wpt // 16, zfill, 0)
        pltpu.sync_copy(zero_v, acc.at[pl.ds(s * wpt, wpt)])
        plsc.subcore_barrier()

        def body(i, carry):
            pltpu.async_copy(ones_v, acc.at[idx_v.at[i]], sem, add=True)
            return carry

        lax.fori_loop(0, rpt, body, 0)

        def drain(i, carry):
            pltpu.make_async_copy(ones_v, acc.at[idx_v.at[0]], sem).wait()
            return carry

        lax.fori_loop(0, rpt, drain, 0)
        plsc.subcore_barrier()

        @pl.when(c == 0)
        def _():
            pltpu.sync_copy(acc.at[pl.ds(s * wpt, wpt)],
                            out0_hbm.at[pl.ds(s * wpt, wpt)])

        @pl.when(c == 1)
        def _():
            pltpu.sync_copy(acc.at[pl.ds(s * wpt, wpt)],
                            out1_hbm.at[pl.ds(s * wpt, wpt)])

    return deg_kernel


_deg = _make_deg()


def _scale_body(deg0_ref, deg1_ref, x_ref, w_ref, out_ref):
    deg = deg0_ref[:, 0] + deg1_ref[:, 0] + 1.0
    dinv = lax.rsqrt(jnp.maximum(deg, 1e-12))
    h = jnp.dot(x_ref[...], w_ref[...], preferred_element_type=jnp.float32)
    row = (lax.broadcasted_iota(jnp.int32, (_BLK, 1), 0)
           + pl.program_id(0) * _BLK)
    out_ref[...] = jnp.where(row < _N, h * dinv[:, None], 0.0)


_BLK = 2048


def _scale(deg0, deg1, x, w):
    grid = _NP // _BLK
    return pl.pallas_call(
        _scale_body,
        out_shape=jax.ShapeDtypeStruct((_NP, _D), jnp.float32),
        grid=(grid,),
        in_specs=[
            pl.BlockSpec((_BLK, 1), lambda i: (i, 0)),
            pl.BlockSpec((_BLK, 1), lambda i: (i, 0)),
            pl.BlockSpec((_BLK, _D), lambda i: (i, 0)),
            pl.BlockSpec((_D, _D), lambda i: (0, 0)),
        ],
        out_specs=pl.BlockSpec((_BLK, _D), lambda i: (i, 0)),
    )(deg0, deg1, x, w)


def _make_edge_agg():
    k = _K
    ch = _CH
    nbuf = 5
    gen = ch // nbuf
    iw = 16
    spt = _NP // _NS

    @functools.partial(
        pl.kernel,
        out_type=jax.ShapeDtypeStruct((_NC, _NP, _D), jnp.float32),
        mesh=_MESH,
        scratch_types=[
            pltpu.VMEM((2, iw, k), jnp.int32),
            pltpu.VMEM((iw, k), jnp.int32),
            [pltpu.VMEM((k, _D), jnp.float32)] * nbuf,
            pltpu.VMEM_SHARED((_NP, _D), jnp.float32),
            [pltpu.SemaphoreType.DMA] * nbuf,
            [pltpu.SemaphoreType.DMA] * nbuf,
        ],
    )
    def edge_agg(scaled_hbm, src_hbm, dst_hbm, out_hbm, src_v, dst_v, rows_v,
                 acc, gsems, ssems):
        c = lax.axis_index("c")
        s = lax.axis_index("s")
        wid = c * _NS + s
        pltpu.sync_copy(src_hbm.at[wid, pl.ds(0, iw)], src_v.at[0])
        pltpu.sync_copy(dst_hbm.at[wid, pl.ds(0, iw)], dst_v)
        pltpu.sync_copy(scaled_hbm.at[pl.ds(s * spt, spt)],
                        acc.at[pl.ds(s * spt, spt)])
        for b in range(nbuf - 1):
            pltpu.async_copy(scaled_hbm.at[src_v.at[0, b]], rows_v[b],
                             gsems[b])
        plsc.subcore_barrier()

        def gwait(b):
            pltpu.make_async_copy(scaled_hbm.at[src_v.at[0, 0]],
                                  rows_v[b], gsems[b]).wait()

        def swait(b):
            pltpu.make_async_copy(rows_v[b], acc.at[dst_v.at[0]],
                                  ssems[b]).wait()

        def body(g, carry):
            for b in range(nbuf):
                i = g * nbuf + b
                pb = (b + nbuf - 1) % nbuf
                gwait(b)

                @pl.when(i > 0)
                def _():
                    swait(pb)

                @pl.when((i % iw == 0) & (i > 0))
                def _():
                    off = pl.multiple_of(i, iw)
                    pltpu.sync_copy(dst_hbm.at[wid, pl.ds(off, iw)], dst_v)

                pltpu.async_copy(rows_v[b], acc.at[dst_v.at[i % iw]],
                                 ssems[b], add=True)
                j = i + nbuf - 1

                @pl.when(j < ch)
                def _():
                    w = (j // iw) % 2

                    @pl.when(j % iw == 0)
                    def _():
                        off = pl.multiple_of(j, iw)
                        pltpu.sync_copy(src_hbm.at[wid, pl.ds(off, iw)],
                                        src_v.at[w])

                    pltpu.async_copy(scaled_hbm.at[src_v.at[w, j % iw]],
                                     rows_v[pb], gsems[pb])

            return carry

        lax.fori_loop(0, gen, body, 0)
        swait((ch - 1) % nbuf)
        plsc.subcore_barrier()
        pltpu.sync_copy(acc.at[pl.ds(s * spt, spt)],
                        out_hbm.at[c, pl.ds(s * spt, spt)])

    return edge_agg


_edge_agg = _make_edge_agg()


def _combine_body(p_ref, scaled_ref, deg0_ref, deg1_ref, b_ref, out_ref):
    deg = deg0_ref[:, 0] + deg1_ref[:, 0] + 1.0
    dinv = lax.rsqrt(jnp.maximum(deg, 1e-12))
    msum = p_ref[0] + p_ref[1] - scaled_ref[...]
    out_ref[...] = jnp.maximum(dinv[:, None] * msum + b_ref[...], 0.0)


def _combine(partials, scaled_p, deg0, deg1, b_gcn2):
    grid = _NP // _BLK
    return pl.pallas_call(
        _combine_body,
        out_shape=jax.ShapeDtypeStruct((_NP, _D), jnp.float32),
        grid=(grid,),
        in_specs=[
            pl.BlockSpec((_NC, _BLK, _D), lambda i: (0, i, 0)),
            pl.BlockSpec((_BLK, _D), lambda i: (i, 0)),
            pl.BlockSpec((_BLK, 1), lambda i: (i, 0)),
            pl.BlockSpec((_BLK, 1), lambda i: (i, 0)),
            pl.BlockSpec((1, _D), lambda i: (0, 0)),
        ],
        out_specs=pl.BlockSpec((_BLK, _D), lambda i: (i, 0)),
    )(partials, scaled_p, deg0, deg1, b_gcn2)


def _make_gather():
    nb = 3 * _B
    k = 128
    rows_all = nb // k
    rpt = rows_all // _NW

    @functools.partial(
        pl.kernel,
        out_type=jax.ShapeDtypeStruct((rows_all, k, _D), jnp.float32),
        mesh=_MESH,
        scratch_types=[
            pltpu.VMEM((rpt, k), jnp.int32),
            pltpu.VMEM((rpt, k, _D), jnp.float32),
            pltpu.SemaphoreType.DMA,
        ],
    )
    def gather_kernel(gcn_hbm, idx_hbm, out_hbm, idx_v, rows_v, sem):
        c = lax.axis_index("c")
        s = lax.axis_index("s")
        wid = c * _NS + s
        pltpu.sync_copy(idx_hbm.at[wid], idx_v)
        for j in range(rpt):
            pltpu.async_copy(gcn_hbm.at[idx_v.at[j]], rows_v.at[j], sem)
        for j in range(rpt):
            pltpu.make_async_copy(gcn_hbm.at[idx_v.at[j]], rows_v.at[j],
                                  sem).wait()
        pltpu.sync_copy(rows_v, out_hbm.at[pl.ds(wid * rpt, rpt)])

    return gather_kernel


_gather = _make_gather()


def _mlp_loss_body(z_ref, w1_ref, b1_ref, w2_ref, b2_ref, out_ref):
    h1 = jnp.dot(z_ref[...], w1_ref[...], preferred_element_type=jnp.float32)
    h1 = jnp.maximum(h1 + b1_ref[...], 0.0)
    o = jnp.dot(h1, w2_ref[...], preferred_element_type=jnp.float32)
    o = o + b2_ref[...]
    a = o[0:_B]
    p = o[_B:2 * _B]
    n = o[2 * _B:3 * _B]
    eps = 1e-6
    d_ap = jnp.sqrt(jnp.sum((a - p + eps) ** 2, axis=1))
    d_an = jnp.sqrt(jnp.sum((a - n + eps) ** 2, axis=1))
    loss = jnp.mean(jnp.maximum(d_ap - d_an + 1.0, 0.0))
    out_ref[...] = loss.reshape(1, 1)


def _mlp_loss(z, w1, b1_2, w2, b2_2):
    return pl.pallas_call(
        _mlp_loss_body,
        out_shape=jax.ShapeDtypeStruct((1, 1), jnp.float32),
        in_specs=[
            pl.BlockSpec((3 * _B, _D), lambda: (0, 0)),
            pl.BlockSpec((_D, _D), lambda: (0, 0)),
            pl.BlockSpec((1, _D), lambda: (0, 0)),
            pl.BlockSpec((_D, _D_OUT), lambda: (0, 0)),
            pl.BlockSpec((1, _D_OUT), lambda: (0, 0)),
        ],
        out_specs=pl.BlockSpec((1, 1), lambda: (0, 0)),
    )(z, w1, b1_2, w2, b2_2)


def kernel(x, W_gcn, b_gcn, W1, b1, W2, b2, edge_index,
           anchor_idx, positive_idx, negative_idx):
    ept = _E // _NW
    srcm = edge_index[0].reshape(_NW, ept)
    dstm = edge_index[1].reshape(_NW, ept)
    pad = jnp.broadcast_to(
        jnp.arange(_N, _N + _EPT - ept, dtype=jnp.int32), (_NW, _EPT - ept))
    src4 = jnp.concatenate([srcm, pad], axis=1).reshape(_NW, _CH, _K)
    dst4 = jnp.concatenate([dstm, pad], axis=1).reshape(_NW, _CH, _K)
    deg0, deg1 = _deg(dst4)
    deg0 = deg0.reshape(_NP, 1)
    deg1 = deg1.reshape(_NP, 1)
    scaled_p = _scale(deg0, deg1, x, W_gcn)
    partials = _edge_agg(scaled_p, src4, dst4)
    gcn = _combine(partials, scaled_p, deg0, deg1, b_gcn.reshape(1, _D))
    idx3 = jnp.concatenate([anchor_idx, positive_idx, negative_idx])
    z = _gather(gcn, idx3.reshape(_NW, 3, 128)).reshape(3 * _B, _D)
    loss = _mlp_loss(z, W1, b1.reshape(1, _D), W2, b2.reshape(1, _D_OUT))
    return loss[0, 0]


# speedup vs baseline: 1.0233x; 1.0045x over previous
import functools

import jax
import jax.numpy as jnp
from jax import lax
from jax.experimental import pallas as pl
from jax.experimental.pallas import tpu as pltpu
from jax.experimental.pallas import tpu_sc as plsc

_N = 10000
_NP = 10240
_E = 320000
_D = 128
_D_OUT = 64
_B = 4096

_INFO = plsc.get_sparse_core_info()
_NC, _NS = _INFO.num_cores, _INFO.num_subcores
_NW = _NC * _NS
_MESH = plsc.VectorSubcoreMesh(core_axis_name="c", subcore_axis_name="s")


_EPT = _NP
_K = 64
_CH = _EPT // _K
_HD = _D // 2


def _make_deg():
    k = _K
    rpt = _CH
    wpt = _NP // _NS

    @functools.partial(
        pl.kernel,
        out_type=(jax.ShapeDtypeStruct((_NP,), jnp.float32),
                  jax.ShapeDtypeStruct((_NP,), jnp.float32)),
        mesh=_MESH,
        scratch_types=[
            pltpu.VMEM((rpt, k), jnp.int32),
            pltpu.VMEM((k,), jnp.float32),
            pltpu.VMEM((wpt,), jnp.float32),
            pltpu.VMEM_SHARED((_NP,), jnp.float32),
            pltpu.SemaphoreType.DMA,
        ],
    )
    def deg_kernel(dst_hbm, out0_hbm, out1_hbm, idx_v, ones_v, zero_v, acc,
                   sem):
        c = lax.axis_index("c")
        s = lax.axis_index("s")
        wid = c * _NS + s
        pltpu.sync_copy(dst_hbm.at[wid], idx_v)

        def fill16(j, carry):
            ones_v[pl.ds(j * 16, 16)] = jnp.ones((16,), jnp.float32)
            return carry

        lax.fori_loop(0, k // 16, fill16, 0)

        def zfill(j, carry):
            zero_v[pl.ds(j * 16, 16)] = jnp.zeros((16,), jnp.float32)
            return carry

        lax.fori_loop(0, wpt // 16, zfill, 0)
        pltpu.sync_copy(zero_v, acc.at[pl.ds(s * wpt, wpt)])
        plsc.subcore_barrier()

        def body(i, carry):
            pltpu.async_copy(ones_v, acc.at[idx_v.at[i]], sem, add=True)
            return carry

        lax.fori_loop(0, rpt, body, 0)

        def drain(i, carry):
            pltpu.make_async_copy(ones_v, acc.at[idx_v.at[0]], sem).wait()
            return carry

        lax.fori_loop(0, rpt, drain, 0)
        plsc.subcore_barrier()

        @pl.when(c == 0)
        def _():
            pltpu.sync_copy(acc.at[pl.ds(s * wpt, wpt)],
                            out0_hbm.at[pl.ds(s * wpt, wpt)])

        @pl.when(c == 1)
        def _():
            pltpu.sync_copy(acc.at[pl.ds(s * wpt, wpt)],
                            out1_hbm.at[pl.ds(s * wpt, wpt)])

    return deg_kernel


_deg = _make_deg()


def _scale_body(deg0_ref, deg1_ref, x_ref, w_ref, out_ref):
    deg = deg0_ref[:, 0] + deg1_ref[:, 0] + 1.0
    dinv = lax.rsqrt(jnp.maximum(deg, 1e-12))
    h = jnp.dot(x_ref[...], w_ref[...], preferred_element_type=jnp.float32)
    row = (lax.broadcasted_iota(jnp.int32, (_BLK, 1), 0)
           + pl.program_id(0) * _BLK)
    out_ref[...] = jnp.where(row < _N, h * dinv[:, None], 0.0)


_BLK = 2048


def _scale(deg0, deg1, x, w):
    grid = _NP // _BLK
    return pl.pallas_call(
        _scale_body,
        out_shape=jax.ShapeDtypeStruct((_NP, _D), jnp.float32),
        grid=(grid,),
        in_specs=[
            pl.BlockSpec((_BLK, 1), lambda i: (i, 0)),
            pl.BlockSpec((_BLK, 1), lambda i: (i, 0)),
            pl.BlockSpec((_BLK, _D), lambda i: (i, 0)),
            pl.BlockSpec((_D, _D), lambda i: (0, 0)),
        ],
        out_specs=pl.BlockSpec((_BLK, _D), lambda i: (i, 0)),
    )(deg0, deg1, x, w)


def _make_edge_agg():
    k = _K
    ch = _CH
    nbuf = 5
    gen = ch // nbuf
    iw = 16
    spt = _NP // _NS

    @functools.partial(
        pl.kernel,
        out_type=jax.ShapeDtypeStruct((_NC, _NP, _D), jnp.float32),
        mesh=_MESH,
        scratch_types=[
            pltpu.VMEM((2, iw, k), jnp.int32),
            pltpu.VMEM((iw, k), jnp.int32),
            [pltpu.VMEM((k, _D), jnp.float32)] * nbuf,
            pltpu.VMEM_SHARED((_NP, _D), jnp.float32),
            [pltpu.SemaphoreType.DMA] * nbuf,
            [pltpu.SemaphoreType.DMA] * nbuf,
        ],
    )
    def edge_agg(scaled_hbm, src_hbm, dst_hbm, out_hbm, src_v, dst_v, rows_v,
                 acc, gsems, ssems):
        c = lax.axis_index("c")
        s = lax.axis_index("s")
        wid = c * _NS + s
        pltpu.sync_copy(src_hbm.at[wid, pl.ds(0, iw)], src_v.at[0])
        pltpu.sync_copy(dst_hbm.at[wid, pl.ds(0, iw)], dst_v)
        pltpu.sync_copy(scaled_hbm.at[pl.ds(s * spt, spt)],
                        acc.at[pl.ds(s * spt, spt)])
        for b in range(nbuf - 1):
            pltpu.async_copy(scaled_hbm.at[src_v.at[0, b]], rows_v[b],
                             gsems[b])
        plsc.subcore_barrier()

        def gwait(b):
            pltpu.make_async_copy(scaled_hbm.at[src_v.at[0, 0]],
                                  rows_v[b], gsems[b]).wait()

        def swait(b):
            pltpu.make_async_copy(rows_v[b], acc.at[dst_v.at[0]],
                                  ssems[b]).wait()

        def body(g, carry):
            for b in range(nbuf):
                i = g * nbuf + b
                pb = (b + nbuf - 1) % nbuf
                gwait(b)

                @pl.when(i > 0)
                def _():
                    swait(pb)

                @pl.when((i % iw == 0) & (i > 0))
                def _():
                    off = pl.multiple_of(i, iw)
                    pltpu.sync_copy(dst_hbm.at[wid, pl.ds(off, iw)], dst_v)

                pltpu.async_copy(rows_v[b], acc.at[dst_v.at[i % iw]],
                                 ssems[b], add=True)
                j = i + nbuf - 1

                @pl.when(j < ch)
                def _():
                    w = (j // iw) % 2

                    @pl.when(j % iw == 0)
                    def _():
                        off = pl.multiple_of(j, iw)
                        pltpu.sync_copy(src_hbm.at[wid, pl.ds(off, iw)],
                                        src_v.at[w])

                    pltpu.async_copy(scaled_hbm.at[src_v.at[w, j % iw]],
                                     rows_v[pb], gsems[pb])

            return carry

        lax.fori_loop(0, gen, body, 0)
        swait((ch - 1) % nbuf)
        plsc.subcore_barrier()
        pltpu.sync_copy(acc.at[pl.ds(s * spt, spt)],
                        out_hbm.at[c, pl.ds(s * spt, spt)])

    return edge_agg


_edge_agg = _make_edge_agg()


def _combine_body(p_ref, scaled_ref, deg0_ref, deg1_ref, b_ref, out_ref):
    deg = deg0_ref[:, 0] + deg1_ref[:, 0] + 1.0
    dinv = lax.rsqrt(jnp.maximum(deg, 1e-12))
    msum = p_ref[0] + p_ref[1] - scaled_ref[...]
    out_ref[...] = jnp.maximum(dinv[:, None] * msum + b_ref[...], 0.0)


def _combine(partials, scaled_p, deg0, deg1, b_gcn2):
    grid = _NP // _BLK
    return pl.pallas_call(
        _combine_body,
        out_shape=jax.ShapeDtypeStruct((_NP, _D), jnp.float32),
        grid=(grid,),
        in_specs=[
            pl.BlockSpec((_NC, _BLK, _D), lambda i: (0, i, 0)),
            pl.BlockSpec((_BLK, _D), lambda i: (i, 0)),
            pl.BlockSpec((_BLK, 1), lambda i: (i, 0)),
            pl.BlockSpec((_BLK, 1), lambda i: (i, 0)),
            pl.BlockSpec((1, _D), lambda i: (0, 0)),
        ],
        out_specs=pl.BlockSpec((_BLK, _D), lambda i: (i, 0)),
    )(partials, scaled_p, deg0, deg1, b_gcn2)


def _make_gather():
    nb = 3 * _B
    k = 128
    rows_all = nb // k
    rpt = rows_all // _NW

    @functools.partial(
        pl.kernel,
        out_type=jax.ShapeDtypeStruct((rows_all, k, _D), jnp.float32),
        mesh=_MESH,
        scratch_types=[
            pltpu.VMEM((rpt, k), jnp.int32),
            pltpu.VMEM((rpt, k, _D), jnp.float32),
            pltpu.SemaphoreType.DMA,
        ],
    )
    def gather_kernel(gcn_hbm, idx_hbm, out_hbm, idx_v, rows_v, sem):
        c = lax.axis_index("c")
        s = lax.axis_index("s")
        wid = c * _NS + s
        pltpu.sync_copy(idx_hbm.at[wid], idx_v)
        for j in range(rpt):
            pltpu.async_copy(gcn_hbm.at[idx_v.at[j]], rows_v.at[j], sem)
        for j in range(rpt):
            pltpu.make_async_copy(gcn_hbm.at[idx_v.at[j]], rows_v.at[j],
                                  sem).wait()
        pltpu.sync_copy(rows_v, out_hbm.at[pl.ds(wid * rpt, rpt)])

    return gather_kernel


_gather = _make_gather()


_MBLK = 2048
_MGRID = _B // _MBLK


def _mlp_loss_body(z_ref, w1_ref, b1_ref, w2_ref, b2_ref, out_ref):
    g = pl.program_id(0)
    zs = z_ref[...]
    z = zs.reshape(3 * _MBLK, _D)
    h1 = jnp.dot(z, w1_ref[...], preferred_element_type=jnp.float32)
    h1 = jnp.maximum(h1 + b1_ref[...], 0.0)
    o = jnp.dot(h1, w2_ref[...], preferred_element_type=jnp.float32)
    o = o + b2_ref[...]
    a = o[0:_MBLK]
    p = o[_MBLK:2 * _MBLK]
    n = o[2 * _MBLK:3 * _MBLK]
    eps = 1e-6
    d_ap = jnp.sqrt(jnp.sum((a - p + eps) ** 2, axis=1))
    d_an = jnp.sqrt(jnp.sum((a - n + eps) ** 2, axis=1))
    part = jnp.sum(jnp.maximum(d_ap - d_an + 1.0, 0.0)) * (1.0 / _B)

    @pl.when(g == 0)
    def _():
        out_ref[...] = jnp.zeros((1, 1), jnp.float32)

    out_ref[...] += part.reshape(1, 1)


def _mlp_loss(z3, w1, b1_2, w2, b2_2):
    return pl.pallas_call(
        _mlp_loss_body,
        out_shape=jax.ShapeDtypeStruct((1, 1), jnp.float32),
        grid=(_MGRID,),
        in_specs=[
            pl.BlockSpec((3, _MBLK, _D), lambda g: (0, g, 0)),
            pl.BlockSpec((_D, _D), lambda g: (0, 0)),
            pl.BlockSpec((1, _D), lambda g: (0, 0)),
            pl.BlockSpec((_D, _D_OUT), lambda g: (0, 0)),
            pl.BlockSpec((1, _D_OUT), lambda g: (0, 0)),
        ],
        out_specs=pl.BlockSpec((1, 1), lambda g: (0, 0)),
    )(z3, w1, b1_2, w2, b2_2)


def kernel(x, W_gcn, b_gcn, W1, b1, W2, b2, edge_index,
           anchor_idx, positive_idx, negative_idx):
    ept = _E // _NW
    srcm = edge_index[0].reshape(_NW, ept)
    dstm = edge_index[1].reshape(_NW, ept)
    pad = jnp.broadcast_to(
        jnp.arange(_N, _N + _EPT - ept, dtype=jnp.int32), (_NW, _EPT - ept))
    src4 = jnp.concatenate([srcm, pad], axis=1).reshape(_NW, _CH, _K)
    dst4 = jnp.concatenate([dstm, pad], axis=1).reshape(_NW, _CH, _K)
    deg0, deg1 = _deg(dst4)
    deg0 = deg0.reshape(_NP, 1)
    deg1 = deg1.reshape(_NP, 1)
    scaled_p = _scale(deg0, deg1, x, W_gcn)
    partials = _edge_agg(scaled_p, src4, dst4)
    gcn = _combine(partials, scaled_p, deg0, deg1, b_gcn.reshape(1, _D))
    idx3 = jnp.concatenate([anchor_idx, positive_idx, negative_idx])
    z3 = _gather(gcn, idx3.reshape(_NW, 3, 128)).reshape(3, _B, _D)
    loss = _mlp_loss(z3, W1, b1.reshape(1, _D), W2, b2.reshape(1, _D_OUT))
    return loss[0, 0]
